# baseline (speedup 1.0000x reference)
import functools

import jax
import jax.numpy as jnp
from jax import lax
from jax.experimental import pallas as pl
from jax.experimental.pallas import tpu as pltpu
from jax.experimental.pallas import tpu_sc as plsc

N = 50000
D = 256
B = 64
NEG = float(jnp.finfo(jnp.float32).min)

NW = 32
CPW = 1568
CPW31 = 1392
RB = 160
TCR = 9040
TCB = TCR // 2
CPWS = 1280
NBLK_S = 8
VPB = RB // 16
PCOLS = 80

_SC_PARAMS = pltpu.CompilerParams(
    use_tc_tiling_on_sc=True, needs_layout_passes=False)


def _mesh():
    return plsc.VectorSubcoreMesh(
        core_axis_name="c", subcore_axis_name="s",
        num_cores=2, num_subcores=16)


@functools.cache
def _make_sc_main():
    @functools.partial(
        pl.kernel,
        mesh=_mesh(),
        out_type=jax.ShapeDtypeStruct((NW, B, PCOLS), jnp.float32),
        scratch_types=[
            pltpu.VMEM((RB, D), jnp.float32),
            pltpu.VMEM((RB, D), jnp.float32),
            pltpu.VMEM((CPWS,), jnp.float32),
            pltpu.VMEM((CPWS,), jnp.int32),
            pltpu.VMEM((16,), jnp.float32),
            pltpu.VMEM((B, PCOLS), jnp.float32),
            pltpu.SemaphoreType.DMA,
            pltpu.SemaphoreType.DMA,
        ],
        compiler_params=_SC_PARAMS,
    )
    def _sc_main(x_hbm, vm_hbm, bi_hbm, id_hbm, po_hbm,
                 buf0, buf1, vmb, bib, idv, pt, sem0, sem1):
        wid = lax.axis_index("s") * 2 + lax.axis_index("c")
        base = TCR + wid * CPWS

        pltpu.sync_copy(id_hbm, idv)
        lane = lax.iota(jnp.int32, 16)
        zero16 = jnp.zeros((16,), jnp.int32)
        negv = jnp.full((16,), NEG, jnp.float32)
        zerofv = jnp.zeros((16,), jnp.float32)
        bigv = jnp.full((16,), N, jnp.int32)

        pltpu.sync_copy(vm_hbm.at[pl.ds(base, CPWS)], vmb)
        pltpu.sync_copy(bi_hbm.at[pl.ds(base, CPWS)], bib)

        def initrow(k, _):
            pt[k, pl.ds(0, 16)] = negv
            pt[k, pl.ds(16, 16)] = zerofv
            pt[k, pl.ds(32, 16)] = plsc.bitcast(bigv, jnp.float32)
            pt[k, pl.ds(48, 16)] = plsc.bitcast(zero16, jnp.float32)
            pt[k, pl.ds(64, 16)] = zerofv
            return 0

        lax.fori_loop(0, B, initrow, 0)

        pltpu.async_copy(x_hbm.at[pl.ds(base, RB)], buf0, sem0)
        pltpu.async_copy(x_hbm.at[pl.ds(base + RB, RB)], buf1, sem1)

        idvec = idv[...]

        def process_vec(jj, v, buf):
            off = jj * RB + v * 16
            vm16 = vmb[pl.ds(off, 16)]
            bi16 = bib[pl.ds(off, 16)]
            rsum = zerofv
            for i in range(16):
                row = v * 16 + i
                acc = buf[row, pl.ds(0, 16)]
                for c in range(1, 16):
                    acc = acc + buf[row, pl.ds(c * 16, 16)]
                tot = jnp.sum(acc)
                rsum = jnp.where(lane == i, tot, rsum)
            col0 = plsc.load_gather(buf, [v * 16 + lane, zero16])
            valid = vm16 > 0.5
            ms_eff = jnp.where(valid, rsum + idvec - col0, NEG)
            rowg = base + off + lane
            mcur = plsc.load_gather(pt, [bi16, lane])
            scur = plsc.load_gather(pt, [bi16, lane + 16])
            wcur = plsc.bitcast(
                plsc.load_gather(pt, [bi16, lane + 32]), jnp.int32)
            ccur = plsc.bitcast(
                plsc.load_gather(pt, [bi16, lane + 48]), jnp.int32)
            better = ms_eff > mcur
            mnew = jnp.maximum(mcur, ms_eff)
            term = jnp.where(valid, jnp.exp(ms_eff - mnew), 0.0)
            snew = scur * jnp.exp(mcur - mnew) + term
            wnew = jnp.where(better, rowg, wcur)
            cnew = ccur + jnp.where(valid, 1, 0)
            plsc.store_scatter(pt, [bi16, lane], mnew)
            plsc.store_scatter(pt, [bi16, lane + 16], snew)
            plsc.store_scatter(pt, [bi16, lane + 32],
                               plsc.bitcast(wnew, jnp.float32))
            plsc.store_scatter(pt, [bi16, lane + 48],
                               plsc.bitcast(cnew, jnp.float32))

        def blockpair(j2, _):
            for b, (buf, sem) in enumerate(((buf0, sem0), (buf1, sem1))):
                jj = 2 * j2 + b
                pltpu.make_async_copy(
                    x_hbm.at[pl.ds(0, RB)], buf, sem).wait()

                def vbody(v, _):
                    process_vec(jj, v, buf)
                    return 0

                lax.fori_loop(0, VPB, vbody, 0)

                nxt = jj + 2

                @pl.when(nxt < NBLK_S)
                def _():
                    pltpu.async_copy(
                        x_hbm.at[pl.ds(base + nxt * RB, RB)], buf, sem)
            return 0

        lax.fori_loop(0, NBLK_S // 2, blockpair, 0)

        @pl.when(wid == NW - 1)
        def _():
            pt[0, pl.ds(64, 16)] = vmb[pl.ds(CPWS - 16, 16)]

        pltpu.sync_copy(pt, po_hbm.at[wid])

    return _sc_main


def _tc_part_body(ident_ref, x_ref, vm_ref, bi_ref,
                  mo_ref, so_ref, wo_ref, co_ref,
                  m_ref, s_ref, w_ref, c_ref):
    i = pl.program_id(0)
    ident = ident_ref[0, 0]

    @pl.when(i == 0)
    def _init():
        m_ref[...] = jnp.full((B, 1), NEG, jnp.float32)
        s_ref[...] = jnp.zeros((B, 1), jnp.float32)
        w_ref[...] = jnp.full((B, 1), N, jnp.int32)
        c_ref[...] = jnp.zeros((B, 1), jnp.int32)

    xb = x_ref[...]
    ones = jnp.ones((1, D), jnp.float32)
    e0 = jnp.where(lax.broadcasted_iota(jnp.int32, (1, D), 1) == 0, 1.0, 0.0)
    dn = (((1,), (1,)), ((), ()))
    rowsum = lax.dot_general(ones, xb, dn,
                             precision=lax.Precision.HIGHEST,
                             preferred_element_type=jnp.float32)
    col0 = lax.dot_general(e0, xb, dn,
                           precision=lax.Precision.HIGHEST,
                           preferred_element_type=jnp.float32)
    vm = vm_ref[0]
    bi = bi_ref[0]
    valid = vm > 0.5
    ms = rowsum + jnp.where(valid, ident - col0, 0.0)
    seg = lax.broadcasted_iota(jnp.int32, (B, TCB), 0)
    hitb = (bi == seg) & valid
    masked = jnp.where(hitb, ms, NEG)
    bmax = jnp.max(masked, axis=1, keepdims=True)
    e = jnp.where(hitb, jnp.exp(masked - bmax), 0.0)
    bsum = jnp.sum(e, axis=1, keepdims=True)
    ri = lax.broadcasted_iota(jnp.int32, (B, TCB), 1) + i * TCB
    bwin = jnp.min(jnp.where(masked >= bmax, ri, N), axis=1, keepdims=True)
    bvc = jnp.sum(jnp.where(hitb, 1, 0), axis=1, keepdims=True)

    m_old = m_ref[...]
    s_old = s_ref[...]
    better = bmax > m_old
    new_m = jnp.maximum(m_old, bmax)
    s_ref[...] = (s_old * jnp.exp(m_old - new_m)
                  + bsum * jnp.exp(bmax - new_m))
    w_ref[...] = jnp.where(better, bwin, w_ref[...])
    c_ref[...] = c_ref[...] + bvc
    m_ref[...] = new_m

    @pl.when(i == (TCR // TCB) - 1)
    def _fin():
        mo_ref[...] = m_ref[...]
        so_ref[...] = s_ref[...]
        wo_ref[...] = w_ref[...]
        co_ref[...] = c_ref[...]


_tc_part = pl.pallas_call(
    _tc_part_body,
    grid=(TCR // TCB,),
    in_specs=[
        pl.BlockSpec(memory_space=pltpu.SMEM),
        pl.BlockSpec((TCB, D), lambda i: (i, 0)),
        pl.BlockSpec((1, 1, TCB), lambda i: (i, 0, 0)),
        pl.BlockSpec((1, 1, TCB), lambda i: (i, 0, 0)),
    ],
    out_specs=[pl.BlockSpec((B, 1), lambda i: (0, 0))] * 4,
    out_shape=[
        jax.ShapeDtypeStruct((B, 1), jnp.float32),
        jax.ShapeDtypeStruct((B, 1), jnp.float32),
        jax.ShapeDtypeStruct((B, 1), jnp.int32),
        jax.ShapeDtypeStruct((B, 1), jnp.int32),
    ],
    scratch_shapes=[
        pltpu.VMEM((B, 1), jnp.float32),
        pltpu.VMEM((B, 1), jnp.float32),
        pltpu.VMEM((B, 1), jnp.int32),
        pltpu.VMEM((B, 1), jnp.int32),
    ],
)


def _merge_body(p_ref, mt_ref, st_ref, wt_ref, ct_ref,
                sel_ref, win_ref, flag_ref):
    p = p_ref[...]
    m4 = p[:, :, 0:16]
    s4 = p[:, :, 16:32]
    w4 = lax.bitcast_convert_type(p[:, :, 32:48], jnp.int32)
    c4 = lax.bitcast_convert_type(p[:, :, 48:64], jnp.int32)
    mstar3 = jnp.max(m4, axis=(0, 2), keepdims=True)
    mstar = mstar3.reshape(B, 1)
    wstar = jnp.min(jnp.where(m4 >= mstar3, w4, N),
                    axis=(0, 2), keepdims=True).reshape(B, 1)
    sstar = jnp.sum(s4 * jnp.exp(m4 - mstar3),
                    axis=(0, 2), keepdims=True).reshape(B, 1)
    cstar = jnp.sum(c4, axis=(0, 2), keepdims=True).reshape(B, 1)
    lastvalid = p[NW - 1, 0, PCOLS - 1] > 0.5
    mtc = mt_ref[...]
    stc = st_ref[...]
    wtc = wt_ref[...]
    ctc = ct_ref[...]
    mall = jnp.maximum(mstar, mtc)
    wall = jnp.minimum(jnp.where(mstar >= mall, wstar, N),
                       jnp.where(mtc >= mall, wtc, N))
    sall = sstar * jnp.exp(mstar - mall) + stc * jnp.exp(mtc - mall)
    call = cstar + ctc
    sel_ref[...] = mall - (mall + jnp.log(sall))
    win_ref[...] = jnp.minimum(wall, N - 1)
    flag_ref[...] = jnp.where((call > 0) | lastvalid, 1, 0)


_tc_merge = pl.pallas_call(
    _merge_body,
    out_shape=[
        jax.ShapeDtypeStruct((B, 1), jnp.float32),
        jax.ShapeDtypeStruct((B, 1), jnp.int32),
        jax.ShapeDtypeStruct((B, 1), jnp.int32),
    ],
)

_RPW = 16
_NGW = B // _RPW


@functools.cache
def _make_finish():
    @functools.partial(
        pl.kernel,
        mesh=_mesh(),
        out_type=[
            jax.ShapeDtypeStruct((B, D), jnp.float32),
            jax.ShapeDtypeStruct((NW, B), jnp.int32),
        ],
        scratch_types=[
            pltpu.VMEM((_RPW,), jnp.int32),
            pltpu.VMEM((_RPW, D), jnp.float32),
            pltpu.VMEM((_RPW,), jnp.int32),
            pltpu.VMEM((16,), jnp.float32),
            pltpu.VMEM((CPW,), jnp.float32),
            pltpu.VMEM((CPW,), jnp.int32),
            pltpu.VMEM((B,), jnp.int32),
            pltpu.VMEM((B, 16), jnp.int32),
            pltpu.VMEM((B,), jnp.int32),
            pltpu.SemaphoreType.DMA,
            pltpu.SemaphoreType.DMA,
        ],
        compiler_params=_SC_PARAMS,
    )
    def _finish(x_hbm, win_hbm, fl_hbm, id_hbm, vm_hbm, bi_hbm,
                out_hbm, rk_hbm,
                idx_v, rows_v, fl_v, idv, vmb, bib, wv, rt, rr, sem, sem2):
        wid = lax.axis_index("s") * 2 + lax.axis_index("c")
        base = wid * CPW
        is31 = wid == NW - 1
        lane = lax.iota(jnp.int32, 16)
        zero16 = jnp.zeros((16,), jnp.int32)

        @pl.when(jnp.logical_not(is31))
        def _():
            pltpu.async_copy(vm_hbm.at[pl.ds(base, CPW)], vmb, sem2)
            pltpu.async_copy(bi_hbm.at[pl.ds(base, CPW)], bib, sem2)

        @pl.when(is31)
        def _():
            pltpu.async_copy(vm_hbm.at[pl.ds(base, CPW31)],
                             vmb.at[pl.ds(0, CPW31)], sem2)
            pltpu.async_copy(bi_hbm.at[pl.ds(base, CPW31)],
                             bib.at[pl.ds(0, CPW31)], sem2)

        pltpu.sync_copy(win_hbm.at[pl.ds(0, B)], wv)

        @pl.when(wid < _NGW)
        def _():
            gbase = pl.multiple_of(wid * _RPW, _RPW)
            pltpu.sync_copy(win_hbm.at[pl.ds(gbase, _RPW)], idx_v)
            pltpu.sync_copy(fl_hbm.at[pl.ds(gbase, _RPW)], fl_v)
            pltpu.sync_copy(id_hbm, idv)
            pltpu.async_copy(x_hbm.at[idx_v], rows_v, sem)

        def zrow(k, _):
            rt[k, pl.ds(0, 16)] = zero16
            return 0

        lax.fori_loop(0, B, zrow, 0)

        nvec = jnp.where(is31, CPW31 // 16, CPW // 16)

        @pl.when(jnp.logical_not(is31))
        def _():
            pltpu.make_async_copy(vm_hbm.at[pl.ds(base, CPW)], vmb,
                                  sem2).wait()
            pltpu.make_async_copy(bi_hbm.at[pl.ds(base, CPW)], bib,
                                  sem2).wait()

        @pl.when(is31)
        def _():
            pltpu.make_async_copy(vm_hbm.at[pl.ds(base, CPW31)],
                                  vmb.at[pl.ds(0, CPW31)], sem2).wait()
            pltpu.make_async_copy(bi_hbm.at[pl.ds(base, CPW31)],
                                  bib.at[pl.ds(0, CPW31)], sem2).wait()

        def rbody(v, _):
            off = v * 16
            vm16 = vmb[pl.ds(off, 16)]
            bi16 = bib[pl.ds(off, 16)]
            rowg = base + off + lane
            w16 = plsc.load_gather(wv, [bi16])
            c = jnp.where((vm16 > 0.5) & (rowg < w16), 1, 0)
            ccur = plsc.load_gather(rt, [bi16, lane])
            plsc.store_scatter(rt, [bi16, lane], ccur + c)
            return 0

        lax.fori_loop(0, nvec, rbody, 0)

        def lred(g, _):
            acc = zero16
            for i in range(16):
                tot = jnp.sum(rt[g * 16 + i])
                acc = jnp.where(lane == i, tot, acc)
            rr[pl.ds(g * 16, 16)] = acc
            return 0

        lax.fori_loop(0, B // 16, lred, 0)
        pltpu.sync_copy(rr, rk_hbm.at[wid])

        @pl.when(wid < _NGW)
        def _():
            gbase = pl.multiple_of(wid * _RPW, _RPW)
            pltpu.make_async_copy(x_hbm.at[idx_v], rows_v, sem).wait()
            rids = lax.iota(jnp.int32, _RPW)
            zcol = jnp.zeros((_RPW,), jnp.int32)
            cur = plsc.load_gather(rows_v, [rids, zcol])
            newc0 = jnp.where(fl_v[...] > 0, idv[...], cur)
            plsc.store_scatter(rows_v, [rids, zcol], newc0)
            pltpu.sync_copy(rows_v, out_hbm.at[pl.ds(gbase, _RPW)])

    return _finish


def kernel(x, agent_E_mask, batch_indices, identifier):
    vm_f = agent_E_mask.astype(jnp.float32)
    bi_i = batch_indices.astype(jnp.int32)
    identv = jnp.broadcast_to(identifier, (16,))
    packed = _make_sc_main()(x, vm_f, bi_i, identv)
    vm_tc = lax.slice(vm_f, (0,), (TCR,)).reshape(TCR // TCB, 1, TCB)
    bi_tc = lax.slice(bi_i, (0,), (TCR,)).reshape(TCR // TCB, 1, TCB)
    mtc, stc, wtc, ctc = _tc_part(
        identifier.reshape(1, 1), x, vm_tc, bi_tc)
    sel, win, flag = _tc_merge(packed, mtc, stc, wtc, ctc)
    hyper, rankp = _make_finish()(
        x, win.reshape(B), flag.reshape(B), identv, vm_f, bi_i)
    act = jnp.sum(rankp, axis=0, dtype=jnp.int32)
    return hyper, act, sel.reshape(B)


# speedup vs baseline: 1.0091x; 1.0091x over previous
import functools

import jax
import jax.numpy as jnp
from jax import lax
from jax.experimental import pallas as pl
from jax.experimental.pallas import tpu as pltpu
from jax.experimental.pallas import tpu_sc as plsc

N = 50000
D = 256
B = 64
NEG = float(jnp.finfo(jnp.float32).min)

NW = 32
CPW = 1568
CPW31 = 1392
RB = 112
TCR = 8528
TCB = TCR // 2
CPWS = 1296
NBLK_S = 12
TAIL_S = 64
VPB = RB // 16
PCOLS = 80

_SC_PARAMS = pltpu.CompilerParams(
    use_tc_tiling_on_sc=True, needs_layout_passes=False)


def _mesh():
    return plsc.VectorSubcoreMesh(
        core_axis_name="c", subcore_axis_name="s",
        num_cores=2, num_subcores=16)


@functools.cache
def _make_sc_main():
    @functools.partial(
        pl.kernel,
        mesh=_mesh(),
        out_type=jax.ShapeDtypeStruct((NW, B, PCOLS), jnp.float32),
        scratch_types=[
            pltpu.VMEM((RB, D), jnp.float32),
            pltpu.VMEM((RB, D), jnp.float32),
            pltpu.VMEM((CPWS,), jnp.float32),
            pltpu.VMEM((CPWS,), jnp.int32),
            pltpu.VMEM((16,), jnp.float32),
            pltpu.VMEM((B, PCOLS), jnp.float32),
            pltpu.SemaphoreType.DMA,
            pltpu.SemaphoreType.DMA,
        ],
        compiler_params=_SC_PARAMS,
    )
    def _sc_main(x_hbm, vm_hbm, bi_hbm, id_hbm, po_hbm,
                 buf0, buf1, vmb, bib, idv, pt, sem0, sem1):
        wid = lax.axis_index("s") * 2 + lax.axis_index("c")
        base = TCR + wid * CPWS

        pltpu.sync_copy(id_hbm, idv)
        lane = lax.iota(jnp.int32, 16)
        zero16 = jnp.zeros((16,), jnp.int32)
        negv = jnp.full((16,), NEG, jnp.float32)
        zerofv = jnp.zeros((16,), jnp.float32)
        bigv = jnp.full((16,), N, jnp.int32)

        pltpu.sync_copy(vm_hbm.at[pl.ds(base, CPWS)], vmb)
        pltpu.sync_copy(bi_hbm.at[pl.ds(base, CPWS)], bib)

        def initrow(k, _):
            pt[k, pl.ds(0, 16)] = negv
            pt[k, pl.ds(16, 16)] = zerofv
            pt[k, pl.ds(32, 16)] = plsc.bitcast(bigv, jnp.float32)
            pt[k, pl.ds(48, 16)] = plsc.bitcast(zero16, jnp.float32)
            pt[k, pl.ds(64, 16)] = zerofv
            return 0

        lax.fori_loop(0, B, initrow, 0)

        pltpu.async_copy(x_hbm.at[pl.ds(base, RB)], buf0, sem0)
        pltpu.async_copy(x_hbm.at[pl.ds(base + RB, RB)], buf1, sem1)

        idvec = idv[...]

        def process_vec(jj, v, buf):
            off = jj * RB + v * 16
            vm16 = vmb[pl.ds(off, 16)]
            bi16 = bib[pl.ds(off, 16)]
            rsum = zerofv
            for i in range(16):
                row = v * 16 + i
                acc = buf[row, pl.ds(0, 16)]
                for c in range(1, 16):
                    acc = acc + buf[row, pl.ds(c * 16, 16)]
                tot = jnp.sum(acc)
                rsum = jnp.where(lane == i, tot, rsum)
            col0 = plsc.load_gather(buf, [v * 16 + lane, zero16])
            valid = vm16 > 0.5
            ms_eff = jnp.where(valid, rsum + idvec - col0, NEG)
            rowg = base + off + lane
            mcur = plsc.load_gather(pt, [bi16, lane])
            scur = plsc.load_gather(pt, [bi16, lane + 16])
            wcur = plsc.bitcast(
                plsc.load_gather(pt, [bi16, lane + 32]), jnp.int32)
            ccur = plsc.bitcast(
                plsc.load_gather(pt, [bi16, lane + 48]), jnp.int32)
            better = ms_eff > mcur
            mnew = jnp.maximum(mcur, ms_eff)
            term = jnp.where(valid, jnp.exp(ms_eff - mnew), 0.0)
            snew = scur * jnp.exp(mcur - mnew) + term
            wnew = jnp.where(better, rowg, wcur)
            cnew = ccur + jnp.where(valid, 1, 0)
            plsc.store_scatter(pt, [bi16, lane], mnew)
            plsc.store_scatter(pt, [bi16, lane + 16], snew)
            plsc.store_scatter(pt, [bi16, lane + 32],
                               plsc.bitcast(wnew, jnp.float32))
            plsc.store_scatter(pt, [bi16, lane + 48],
                               plsc.bitcast(cnew, jnp.float32))

        def blockpair(j2, _):
            for b, (buf, sem) in enumerate(((buf0, sem0), (buf1, sem1))):
                jj = 2 * j2 + b
                partial = jj == NBLK_S - 1

                @pl.when(jnp.logical_not(partial))
                def _():
                    pltpu.make_async_copy(
                        x_hbm.at[pl.ds(0, RB)], buf, sem).wait()

                @pl.when(partial)
                def _():
                    pltpu.make_async_copy(
                        x_hbm.at[pl.ds(0, TAIL_S)],
                        buf.at[pl.ds(0, TAIL_S)], sem).wait()

                nv = jnp.where(partial, TAIL_S // 16, VPB)

                def vbody(v, _):
                    process_vec(jj, v, buf)
                    return 0

                lax.fori_loop(0, nv, vbody, 0)

                nxt = jj + 2
                nxt_full = nxt < NBLK_S - 1
                nxt_part = nxt == NBLK_S - 1

                @pl.when(nxt_full)
                def _():
                    pltpu.async_copy(
                        x_hbm.at[pl.ds(base + nxt * RB, RB)], buf, sem)

                @pl.when(nxt_part)
                def _():
                    pltpu.async_copy(
                        x_hbm.at[pl.ds(base + nxt * RB, TAIL_S)],
                        buf.at[pl.ds(0, TAIL_S)], sem)
            return 0

        lax.fori_loop(0, NBLK_S // 2, blockpair, 0)

        @pl.when(wid == NW - 1)
        def _():
            pt[0, pl.ds(64, 16)] = vmb[pl.ds(CPWS - 16, 16)]

        pltpu.sync_copy(pt, po_hbm.at[wid])

    return _sc_main


def _tc_part_body(ident_ref, x_ref, vm_ref, bi_ref,
                  mo_ref, so_ref, wo_ref, co_ref,
                  m_ref, s_ref, w_ref, c_ref):
    i = pl.program_id(0)
    ident = ident_ref[0, 0]

    @pl.when(i == 0)
    def _init():
        m_ref[...] = jnp.full((B, 1), NEG, jnp.float32)
        s_ref[...] = jnp.zeros((B, 1), jnp.float32)
        w_ref[...] = jnp.full((B, 1), N, jnp.int32)
        c_ref[...] = jnp.zeros((B, 1), jnp.int32)

    xb = x_ref[...]
    ones = jnp.ones((1, D), jnp.float32)
    e0 = jnp.where(lax.broadcasted_iota(jnp.int32, (1, D), 1) == 0, 1.0, 0.0)
    dn = (((1,), (1,)), ((), ()))
    rowsum = lax.dot_general(ones, xb, dn,
                             precision=lax.Precision.HIGHEST,
                             preferred_element_type=jnp.float32)
    col0 = lax.dot_general(e0, xb, dn,
                           precision=lax.Precision.HIGHEST,
                           preferred_element_type=jnp.float32)
    vm = vm_ref[0]
    bi = bi_ref[0]
    valid = vm > 0.5
    ms = rowsum + jnp.where(valid, ident - col0, 0.0)
    seg = lax.broadcasted_iota(jnp.int32, (B, TCB), 0)
    hitb = (bi == seg) & valid
    masked = jnp.where(hitb, ms, NEG)
    bmax = jnp.max(masked, axis=1, keepdims=True)
    e = jnp.where(hitb, jnp.exp(masked - bmax), 0.0)
    bsum = jnp.sum(e, axis=1, keepdims=True)
    ri = lax.broadcasted_iota(jnp.int32, (B, TCB), 1) + i * TCB
    bwin = jnp.min(jnp.where(masked >= bmax, ri, N), axis=1, keepdims=True)
    bvc = jnp.sum(jnp.where(hitb, 1, 0), axis=1, keepdims=True)

    m_old = m_ref[...]
    s_old = s_ref[...]
    better = bmax > m_old
    new_m = jnp.maximum(m_old, bmax)
    s_ref[...] = (s_old * jnp.exp(m_old - new_m)
                  + bsum * jnp.exp(bmax - new_m))
    w_ref[...] = jnp.where(better, bwin, w_ref[...])
    c_ref[...] = c_ref[...] + bvc
    m_ref[...] = new_m

    @pl.when(i == (TCR // TCB) - 1)
    def _fin():
        mo_ref[...] = m_ref[...]
        so_ref[...] = s_ref[...]
        wo_ref[...] = w_ref[...]
        co_ref[...] = c_ref[...]


_tc_part = pl.pallas_call(
    _tc_part_body,
    grid=(TCR // TCB,),
    in_specs=[
        pl.BlockSpec(memory_space=pltpu.SMEM),
        pl.BlockSpec((TCB, D), lambda i: (i, 0)),
        pl.BlockSpec((1, 1, TCB), lambda i: (i, 0, 0)),
        pl.BlockSpec((1, 1, TCB), lambda i: (i, 0, 0)),
    ],
    out_specs=[pl.BlockSpec((B, 1), lambda i: (0, 0))] * 4,
    out_shape=[
        jax.ShapeDtypeStruct((B, 1), jnp.float32),
        jax.ShapeDtypeStruct((B, 1), jnp.float32),
        jax.ShapeDtypeStruct((B, 1), jnp.int32),
        jax.ShapeDtypeStruct((B, 1), jnp.int32),
    ],
    scratch_shapes=[
        pltpu.VMEM((B, 1), jnp.float32),
        pltpu.VMEM((B, 1), jnp.float32),
        pltpu.VMEM((B, 1), jnp.int32),
        pltpu.VMEM((B, 1), jnp.int32),
    ],
)


def _merge_body(p_ref, mt_ref, st_ref, wt_ref, ct_ref,
                sel_ref, win_ref, flag_ref):
    p = p_ref[...]
    m4 = p[:, :, 0:16]
    s4 = p[:, :, 16:32]
    w4 = lax.bitcast_convert_type(p[:, :, 32:48], jnp.int32)
    c4 = lax.bitcast_convert_type(p[:, :, 48:64], jnp.int32)
    mstar3 = jnp.max(m4, axis=(0, 2), keepdims=True)
    mstar = mstar3.reshape(B, 1)
    wstar = jnp.min(jnp.where(m4 >= mstar3, w4, N),
                    axis=(0, 2), keepdims=True).reshape(B, 1)
    sstar = jnp.sum(s4 * jnp.exp(m4 - mstar3),
                    axis=(0, 2), keepdims=True).reshape(B, 1)
    cstar = jnp.sum(c4, axis=(0, 2), keepdims=True).reshape(B, 1)
    lastvalid = p[NW - 1, 0, PCOLS - 1] > 0.5
    mtc = mt_ref[...]
    stc = st_ref[...]
    wtc = wt_ref[...]
    ctc = ct_ref[...]
    mall = jnp.maximum(mstar, mtc)
    wall = jnp.minimum(jnp.where(mstar >= mall, wstar, N),
                       jnp.where(mtc >= mall, wtc, N))
    sall = sstar * jnp.exp(mstar - mall) + stc * jnp.exp(mtc - mall)
    call = cstar + ctc
    sel_ref[...] = mall - (mall + jnp.log(sall))
    win_ref[...] = jnp.minimum(wall, N - 1)
    flag_ref[...] = jnp.where((call > 0) | lastvalid, 1, 0)


_tc_merge = pl.pallas_call(
    _merge_body,
    out_shape=[
        jax.ShapeDtypeStruct((B, 1), jnp.float32),
        jax.ShapeDtypeStruct((B, 1), jnp.int32),
        jax.ShapeDtypeStruct((B, 1), jnp.int32),
    ],
)

_RPW = 16
_NGW = B // _RPW


@functools.cache
def _make_finish():
    @functools.partial(
        pl.kernel,
        mesh=_mesh(),
        out_type=[
            jax.ShapeDtypeStruct((B, D), jnp.float32),
            jax.ShapeDtypeStruct((NW, B), jnp.int32),
        ],
        scratch_types=[
            pltpu.VMEM((_RPW,), jnp.int32),
            pltpu.VMEM((_RPW, D), jnp.float32),
            pltpu.VMEM((_RPW,), jnp.int32),
            pltpu.VMEM((16,), jnp.float32),
            pltpu.VMEM((CPW,), jnp.float32),
            pltpu.VMEM((CPW,), jnp.int32),
            pltpu.VMEM((B,), jnp.int32),
            pltpu.VMEM((B, 16), jnp.int32),
            pltpu.VMEM((B,), jnp.int32),
            pltpu.SemaphoreType.DMA,
            pltpu.SemaphoreType.DMA,
        ],
        compiler_params=_SC_PARAMS,
    )
    def _finish(x_hbm, win_hbm, fl_hbm, id_hbm, vm_hbm, bi_hbm,
                out_hbm, rk_hbm,
                idx_v, rows_v, fl_v, idv, vmb, bib, wv, rt, rr, sem, sem2):
        wid = lax.axis_index("s") * 2 + lax.axis_index("c")
        base = wid * CPW
        is31 = wid == NW - 1
        lane = lax.iota(jnp.int32, 16)
        zero16 = jnp.zeros((16,), jnp.int32)

        @pl.when(jnp.logical_not(is31))
        def _():
            pltpu.async_copy(vm_hbm.at[pl.ds(base, CPW)], vmb, sem2)
            pltpu.async_copy(bi_hbm.at[pl.ds(base, CPW)], bib, sem2)

        @pl.when(is31)
        def _():
            pltpu.async_copy(vm_hbm.at[pl.ds(base, CPW31)],
                             vmb.at[pl.ds(0, CPW31)], sem2)
            pltpu.async_copy(bi_hbm.at[pl.ds(base, CPW31)],
                             bib.at[pl.ds(0, CPW31)], sem2)

        pltpu.sync_copy(win_hbm.at[pl.ds(0, B)], wv)

        @pl.when(wid < _NGW)
        def _():
            gbase = pl.multiple_of(wid * _RPW, _RPW)
            pltpu.sync_copy(win_hbm.at[pl.ds(gbase, _RPW)], idx_v)
            pltpu.sync_copy(fl_hbm.at[pl.ds(gbase, _RPW)], fl_v)
            pltpu.sync_copy(id_hbm, idv)
            pltpu.async_copy(x_hbm.at[idx_v], rows_v, sem)

        def zrow(k, _):
            rt[k, pl.ds(0, 16)] = zero16
            return 0

        lax.fori_loop(0, B, zrow, 0)

        nvec = jnp.where(is31, CPW31 // 16, CPW // 16)

        @pl.when(jnp.logical_not(is31))
        def _():
            pltpu.make_async_copy(vm_hbm.at[pl.ds(base, CPW)], vmb,
                                  sem2).wait()
            pltpu.make_async_copy(bi_hbm.at[pl.ds(base, CPW)], bib,
                                  sem2).wait()

        @pl.when(is31)
        def _():
            pltpu.make_async_copy(vm_hbm.at[pl.ds(base, CPW31)],
                                  vmb.at[pl.ds(0, CPW31)], sem2).wait()
            pltpu.make_async_copy(bi_hbm.at[pl.ds(base, CPW31)],
                                  bib.at[pl.ds(0, CPW31)], sem2).wait()

        def rbody(v, _):
            off = v * 16
            vm16 = vmb[pl.ds(off, 16)]
            bi16 = bib[pl.ds(off, 16)]
            rowg = base + off + lane
            w16 = plsc.load_gather(wv, [bi16])
            c = jnp.where((vm16 > 0.5) & (rowg < w16), 1, 0)
            ccur = plsc.load_gather(rt, [bi16, lane])
            plsc.store_scatter(rt, [bi16, lane], ccur + c)
            return 0

        lax.fori_loop(0, nvec, rbody, 0)

        def lred(g, _):
            acc = zero16
            for i in range(16):
                tot = jnp.sum(rt[g * 16 + i])
                acc = jnp.where(lane == i, tot, acc)
            rr[pl.ds(g * 16, 16)] = acc
            return 0

        lax.fori_loop(0, B // 16, lred, 0)
        pltpu.sync_copy(rr, rk_hbm.at[wid])

        @pl.when(wid < _NGW)
        def _():
            gbase = pl.multiple_of(wid * _RPW, _RPW)
            pltpu.make_async_copy(x_hbm.at[idx_v], rows_v, sem).wait()
            rids = lax.iota(jnp.int32, _RPW)
            zcol = jnp.zeros((_RPW,), jnp.int32)
            cur = plsc.load_gather(rows_v, [rids, zcol])
            newc0 = jnp.where(fl_v[...] > 0, idv[...], cur)
            plsc.store_scatter(rows_v, [rids, zcol], newc0)
            pltpu.sync_copy(rows_v, out_hbm.at[pl.ds(gbase, _RPW)])

    return _finish


def kernel(x, agent_E_mask, batch_indices, identifier):
    vm_f = agent_E_mask.astype(jnp.float32)
    bi_i = batch_indices.astype(jnp.int32)
    identv = jnp.broadcast_to(identifier, (16,))
    packed = _make_sc_main()(x, vm_f, bi_i, identv)
    vm_tc = lax.slice(vm_f, (0,), (TCR,)).reshape(TCR // TCB, 1, TCB)
    bi_tc = lax.slice(bi_i, (0,), (TCR,)).reshape(TCR // TCB, 1, TCB)
    mtc, stc, wtc, ctc = _tc_part(
        identifier.reshape(1, 1), x, vm_tc, bi_tc)
    sel, win, flag = _tc_merge(packed, mtc, stc, wtc, ctc)
    hyper, rankp = _make_finish()(
        x, win.reshape(B), flag.reshape(B), identv, vm_f, bi_i)
    act = jnp.sum(rankp, axis=0, dtype=jnp.int32)
    return hyper, act, sel.reshape(B)
